# baseline (speedup 1.0000x reference)
import functools

import jax
import jax.numpy as jnp
import numpy as np
from jax import lax
from jax.experimental import pallas as pl
from jax.experimental.pallas import tpu as pltpu
from jax.experimental.pallas import tpu_sc as plsc

N = 10000
E = 320000
DF = 128
H = 64
K = 16
NC_OUT = 8

NCORES = 2
NSUB = 16
NW = NCORES * NSUB
CHUNK = 128
NCHUNK = 80
EPAD = NW * NCHUNK * CHUNK
NPAD_DEG = 10240
ROWS_PER_TILE = NPAD_DEG // NSUB
NPAD = 10048
ROWS_AGG = NPAD // NSUB

_mesh = plsc.VectorSubcoreMesh(core_axis_name="c", subcore_axis_name="s")


@functools.partial(
    pl.kernel,
    out_type=[jax.ShapeDtypeStruct((NPAD_DEG,), jnp.float32),
              jax.ShapeDtypeStruct((NPAD_DEG,), jnp.float32)],
    mesh=_mesh,
    scratch_types=[
        pltpu.VMEM((NCHUNK, CHUNK), jnp.int32),
        pltpu.VMEM((CHUNK,), jnp.float32),
        pltpu.VMEM((ROWS_PER_TILE,), jnp.float32),
        pltpu.VMEM_SHARED((NPAD_DEG,), jnp.float32),
        pltpu.SemaphoreType.DMA,
    ],
)
def _deg_kernel(dst_hbm, out0_hbm, out1_hbm, idx_v, ones_v, zrow_v, shared_deg, sem):
    cid = lax.axis_index("c")
    sid = lax.axis_index("s")
    wid = cid * NSUB + sid
    for i in range(CHUNK // 16):
        ones_v[pl.ds(16 * i, 16)] = jnp.full((16,), 1.0, jnp.float32)
    for i in range(ROWS_PER_TILE // 16):
        zrow_v[pl.ds(16 * i, 16)] = jnp.zeros((16,), jnp.float32)
    pltpu.sync_copy(zrow_v,
                    shared_deg.at[pl.ds(sid * ROWS_PER_TILE, ROWS_PER_TILE)])
    pltpu.sync_copy(dst_hbm.at[wid], idx_v)
    plsc.subcore_barrier()

    def body(j, carry):
        pltpu.sync_copy(ones_v, shared_deg.at[idx_v.at[j]], add=True)
        return carry

    lax.fori_loop(0, NCHUNK, body, 0)
    plsc.subcore_barrier()
    tile_rows = pl.ds(sid * ROWS_PER_TILE, ROWS_PER_TILE)

    @pl.when(cid == 0)
    def _():
        pltpu.sync_copy(shared_deg.at[tile_rows], out0_hbm.at[tile_rows])

    @pl.when(cid == 1)
    def _():
        pltpu.sync_copy(shared_deg.at[tile_rows], out1_hbm.at[tile_rows])


@functools.partial(
    pl.kernel,
    out_type=jax.ShapeDtypeStruct((NCORES, NPAD, DF), jnp.float32),
    mesh=_mesh,
    scratch_types=(
        [pltpu.VMEM((1, CHUNK), jnp.int32)] * 8
        + [pltpu.VMEM((CHUNK, DF // 2), jnp.int32)] * 2
        + [pltpu.VMEM((CHUNK, DF), jnp.float32)] * 2
        + [pltpu.SemaphoreType.DMA] * 8
        + [pltpu.VMEM_SHARED((NPAD, DF), jnp.float32)]
    ),
    compiler_params=pltpu.CompilerParams(use_tc_tiling_on_sc=False),
)
def _agg_kernel(xs_hbm, src_hbm, dst_hbm, zeros_hbm, out_hbm,
                ibs0, ibd0, ibs1, ibd1, ibs2, ibd2, ibs3, ibd3,
                gbuf0, gbuf1, sbuf0, sbuf1,
                semg0, semg1, sems0, sems1, semi0, semi1, semi2, semi3,
                shared_agg):
    cid = lax.axis_index("c")
    sid = lax.axis_index("s")
    wid = cid * NSUB + sid
    ibs = (ibs0, ibs1, ibs2, ibs3)
    ibd = (ibd0, ibd1, ibd2, ibd3)
    semi = (semi0, semi1, semi2, semi3)
    gbufs = (gbuf0, gbuf1)
    sbufs = (sbuf0, sbuf1)
    semgs = (semg0, semg1)
    semss = (sems0, sems1)
    span = pl.ds(pl.multiple_of((sid // 2) * (2 * ROWS_AGG), 8), 2 * ROWS_AGG)

    @pl.when(sid % 2 == 0)
    def _():
        pltpu.sync_copy(zeros_hbm.at[span], shared_agg.at[span])

    plsc.subcore_barrier()

    def fetch_idx(c, p):
        pltpu.async_copy(src_hbm.at[wid, c], ibs[p].at[0], semi[p])
        pltpu.async_copy(dst_hbm.at[wid, c], ibd[p].at[0], semi[p])

    def wait_idx(c, p):
        pltpu.make_async_copy(src_hbm.at[wid, c], ibs[p].at[0], semi[p]).wait()
        pltpu.make_async_copy(dst_hbm.at[wid, c], ibd[p].at[0], semi[p]).wait()

    fetch_idx(0, 0)
    fetch_idx(1, 1)

    def unpack_chunk(gb, sb):
        def conv_body(r, carry):
            for g in range(4):
                v = gb[r, pl.ds(16 * g, 16)]
                sb[r, pl.ds(32 * g, 16)] = lax.bitcast_convert_type(v << 16, jnp.float32)
                sb[r, pl.ds(32 * g + 16, 16)] = lax.bitcast_convert_type(
                    v & jnp.int32(-65536), jnp.float32)
            return carry

        lax.fori_loop(0, CHUNK, conv_body, 0)

    def slot(c, b, p):
        bo = 1 - b
        p2 = (p + 2) % 4

        @pl.when(jnp.logical_and(c >= 2, c < NCHUNK + 2))
        def _():
            pltpu.make_async_copy(
                sbufs[b], shared_agg.at[ibd[p2].at[0]], semss[b]).wait()

        @pl.when(c + 2 < NCHUNK)
        def _():
            fetch_idx(c + 2, p2)

        @pl.when(c < NCHUNK)
        def _():
            wait_idx(c, p)
            pltpu.async_copy(xs_hbm.at[ibs[p].at[0]], gbufs[b], semgs[b])

        @pl.when(jnp.logical_and(c >= 1, c < NCHUNK + 1))
        def _():
            p1 = (p + 3) % 4
            pltpu.make_async_copy(
                xs_hbm.at[ibs[p1].at[0]], gbufs[bo], semgs[bo]).wait()
            unpack_chunk(gbufs[bo], sbufs[bo])
            pltpu.async_copy(
                sbufs[bo], shared_agg.at[ibd[p1].at[0]], semss[bo], add=True)

    def body(jj, carry):
        for u in range(4):
            c = 4 * jj + u
            slot(c, u % 2, u)
        return carry

    lax.fori_loop(0, (NCHUNK + 4) // 4, body, 0)
    plsc.subcore_barrier()

    @pl.when(sid % 2 == 0)
    def _():
        pltpu.sync_copy(shared_agg.at[span], out_hbm.at[cid, span])


def _prep_body(deg_ref, x_ref, xs_ref, dinv_ref, dnorm_ref):
    deg = (deg_ref[0] + deg_ref[1])[:NPAD]
    dinv = lax.rsqrt(deg + 1.0)
    dinv_ref[...] = dinv
    maxdeg = jnp.max(deg[:N])
    dnorm_ref[...] = deg / jnp.maximum(maxdeg, 1.0)
    xs_ref[:N] = (x_ref[...] * dinv[:N]).astype(jnp.bfloat16)
    xs_ref[N:] = jnp.zeros((NPAD - N, DF), jnp.bfloat16)


def _prep_call(deg3, x):
    return pl.pallas_call(
        _prep_body,
        out_shape=[
            jax.ShapeDtypeStruct((NPAD, DF), jnp.bfloat16),
            jax.ShapeDtypeStruct((NPAD, 1), jnp.float32),
            jax.ShapeDtypeStruct((NPAD, 1), jnp.float32),
        ],
    )(deg3, x)


def _dense_body(agg_ref, x_ref, dinv_ref, dnorm_ref, w1_ref, b1_ref, w2_ref,
                b2_ref, tf_ref, tc_ref, alpha_ref, gx_ref, gy_ref, bx_ref,
                by_ref, lwx_ref, lwy_ref, lb_ref, out_ref):
    f32 = jnp.float32
    agg = agg_ref[0] + agg_ref[1]
    dinv = dinv_ref[...]
    sloop = jnp.concatenate(
        [x_ref[...] * (dinv[:N] * dinv[:N]),
         jnp.zeros((NPAD - N, DF), f32)], axis=0)
    conv = dinv * agg + sloop
    h1 = jnp.maximum(
        jnp.dot(conv, w1_ref[...], preferred_element_type=f32) + b1_ref[...], 0.0)
    x2 = jnp.maximum(
        jnp.dot(conv, w2_ref[...], preferred_element_type=f32) + b2_ref[...], 0.0)

    tf = tf_ref[...]
    qf = jnp.mean(tf, axis=1)
    qf2 = jnp.mean(jnp.sum(tf * tf, axis=2), axis=1)
    sk = jnp.mean(tc_ref[...], axis=(1, 2))

    xx = jnp.sum(h1 * h1, axis=1, keepdims=True)
    cross = lax.dot_general(h1, qf, (((1,), (1,)), ((), ())),
                            preferred_element_type=f32)
    feat = xx + qf2[None, :] - 2.0 * cross
    struct = (dnorm_ref[...] - sk[None, :]) ** 2
    alpha = jax.nn.sigmoid(alpha_ref[0, 0])
    y = alpha * feat + (1.0 - alpha) * struct

    n = jnp.float32(N)
    m64 = lax.broadcasted_iota(jnp.int32, (NPAD, H), 0) < N
    m16 = lax.broadcasted_iota(jnp.int32, (NPAD, K), 0) < N
    mean_x = jnp.sum(jnp.where(m64, x2, 0.0), axis=0, keepdims=True) / n
    dx = jnp.where(m64, x2 - mean_x, 0.0)
    var_x = jnp.sum(dx * dx, axis=0, keepdims=True) / n
    mean_y = jnp.sum(jnp.where(m16, y, 0.0), axis=0, keepdims=True) / n
    dy = jnp.where(m16, y - mean_y, 0.0)
    var_y = jnp.sum(dy * dy, axis=0, keepdims=True) / n

    zx = (x2 - mean_x) * lax.rsqrt(var_x + 1e-5) * gx_ref[...] + bx_ref[...]
    zy = (y - mean_y) * lax.rsqrt(var_y + 1e-5) * gy_ref[...] + by_ref[...]
    res = (jnp.dot(zx, lwx_ref[...], preferred_element_type=f32)
           + jnp.dot(zy, lwy_ref[...], preferred_element_type=f32)
           + lb_ref[...])
    out_ref[...] = res[:N]


def _dense_call(*args):
    return pl.pallas_call(
        _dense_body,
        out_shape=jax.ShapeDtypeStruct((N, NC_OUT), jnp.float32),
    )(*args)


def kernel(x, edge_index, W1, b1, W2, b2, templates_F, templates_C,
           alpha_p, bn_gamma, bn_beta, lin_W, lin_b):
    f32 = jnp.float32
    pad = EPAD - E
    dummy = N + (jnp.arange(pad, dtype=jnp.int32) % 16)
    srcp = jnp.concatenate([edge_index[0], dummy]).reshape(NW, NCHUNK, CHUNK)
    dstp = jnp.concatenate([edge_index[1], dummy]).reshape(NW, NCHUNK, CHUNK)

    deg0, deg1 = _deg_kernel(dstp)
    deg3 = jnp.stack([deg0, deg1]).reshape(NCORES, NPAD_DEG, 1)
    xs_bf, dinv, dnorm = _prep_call(deg3, x)
    perm = np.empty((DF,), np.int32)
    for b in range(4):
        for t in range(16):
            perm[32 * b + 2 * t] = 32 * b + t
            perm[32 * b + 2 * t + 1] = 32 * b + 16 + t
    xs_pk = lax.bitcast_convert_type(
        xs_bf[:, perm].reshape(NPAD, DF // 2, 2), jnp.int32)
    zeros = jnp.zeros((NPAD, DF), f32)
    agg_parts = _agg_kernel(xs_pk, srcp, dstp, zeros)

    out = _dense_call(
        agg_parts, x, dinv, dnorm,
        W1, b1.reshape(1, H), W2, b2.reshape(1, H),
        templates_F, templates_C, alpha_p.reshape(1, 1),
        bn_gamma[:H].reshape(1, H), bn_gamma[H:].reshape(1, K),
        bn_beta[:H].reshape(1, H), bn_beta[H:].reshape(1, K),
        lin_W[:H], lin_W[H:], lin_b.reshape(1, NC_OUT),
    )
    return out


# speedup vs baseline: 1.9856x; 1.9856x over previous
import functools

import jax
import jax.numpy as jnp
import numpy as np
from jax import lax
from jax.experimental import pallas as pl
from jax.experimental.pallas import tpu as pltpu
from jax.experimental.pallas import tpu_sc as plsc

N = 10000
E = 320000
DF = 128
H = 64
K = 16
NC_OUT = 8

NCORES = 2
NSUB = 16
NW = NCORES * NSUB
CHUNK = 128
NCHUNK = 80
EPAD = NW * NCHUNK * CHUNK
NPAD_DEG = 10240
ROWS_PER_TILE = NPAD_DEG // NSUB
NPAD = 10048
ROWS_AGG = NPAD // NSUB

_mesh = plsc.VectorSubcoreMesh(core_axis_name="c", subcore_axis_name="s")


@functools.partial(
    pl.kernel,
    out_type=[jax.ShapeDtypeStruct((NPAD_DEG,), jnp.float32),
              jax.ShapeDtypeStruct((NPAD_DEG,), jnp.float32)],
    mesh=_mesh,
    scratch_types=[
        pltpu.VMEM((NCHUNK, CHUNK), jnp.int32),
        pltpu.VMEM((CHUNK,), jnp.float32),
        pltpu.VMEM((ROWS_PER_TILE,), jnp.float32),
        pltpu.VMEM_SHARED((NPAD_DEG,), jnp.float32),
        pltpu.SemaphoreType.DMA,
    ],
)
def _deg_kernel(dst_hbm, out0_hbm, out1_hbm, idx_v, ones_v, zrow_v, shared_deg, sem):
    cid = lax.axis_index("c")
    sid = lax.axis_index("s")
    wid = cid * NSUB + sid
    for i in range(CHUNK // 16):
        ones_v[pl.ds(16 * i, 16)] = jnp.full((16,), 1.0, jnp.float32)
    for i in range(ROWS_PER_TILE // 16):
        zrow_v[pl.ds(16 * i, 16)] = jnp.zeros((16,), jnp.float32)
    pltpu.sync_copy(zrow_v,
                    shared_deg.at[pl.ds(sid * ROWS_PER_TILE, ROWS_PER_TILE)])
    pltpu.sync_copy(dst_hbm.at[wid], idx_v)
    plsc.subcore_barrier()

    def body(j, carry):
        pltpu.sync_copy(ones_v, shared_deg.at[idx_v.at[j]], add=True)
        return carry

    lax.fori_loop(0, NCHUNK, body, 0)
    plsc.subcore_barrier()
    tile_rows = pl.ds(sid * ROWS_PER_TILE, ROWS_PER_TILE)

    @pl.when(cid == 0)
    def _():
        pltpu.sync_copy(shared_deg.at[tile_rows], out0_hbm.at[tile_rows])

    @pl.when(cid == 1)
    def _():
        pltpu.sync_copy(shared_deg.at[tile_rows], out1_hbm.at[tile_rows])


@functools.partial(
    pl.kernel,
    out_type=jax.ShapeDtypeStruct((NCORES, NPAD, DF), jnp.float32),
    mesh=_mesh,
    scratch_types=(
        [pltpu.VMEM((1, CHUNK), jnp.int32)] * 10
        + [pltpu.VMEM((CHUNK, DF), jnp.float32)] * 3
        + [pltpu.SemaphoreType.DMA] * 11
        + [pltpu.VMEM_SHARED((NPAD, DF), jnp.float32)]
    ),
)
def _agg_kernel(xs_hbm, src_hbm, dst_hbm, zeros_hbm, out_hbm,
                ibs0, ibd0, ibs1, ibd1, ibs2, ibd2, ibs3, ibd3, ibs4, ibd4,
                gbuf0, gbuf1, gbuf2,
                semg0, semg1, semg2, sems0, sems1, sems2,
                semi0, semi1, semi2, semi3, semi4,
                shared_agg):
    cid = lax.axis_index("c")
    sid = lax.axis_index("s")
    wid = cid * NSUB + sid
    ibs = (ibs0, ibs1, ibs2, ibs3, ibs4)
    ibd = (ibd0, ibd1, ibd2, ibd3, ibd4)
    semi = (semi0, semi1, semi2, semi3, semi4)
    gbufs = (gbuf0, gbuf1, gbuf2)
    semgs = (semg0, semg1, semg2)
    semss = (sems0, sems1, sems2)
    span = pl.ds(pl.multiple_of((sid // 2) * (2 * ROWS_AGG), 8), 2 * ROWS_AGG)

    @pl.when(sid % 2 == 0)
    def _():
        pltpu.sync_copy(zeros_hbm.at[span], shared_agg.at[span])

    plsc.subcore_barrier()

    def fetch_idx(c, p):
        pltpu.async_copy(src_hbm.at[wid, c], ibs[p].at[0], semi[p])
        pltpu.async_copy(dst_hbm.at[wid, c], ibd[p].at[0], semi[p])

    def wait_idx(c, p):
        pltpu.make_async_copy(src_hbm.at[wid, c], ibs[p].at[0], semi[p]).wait()
        pltpu.make_async_copy(dst_hbm.at[wid, c], ibd[p].at[0], semi[p]).wait()

    fetch_idx(0, 0)
    fetch_idx(1, 1)

    def slot(c, b, p):
        b2 = (b + 1) % 3
        p2 = (p + 2) % 5
        p1 = (p + 3) % 5

        @pl.when(jnp.logical_and(c >= 3, c < NCHUNK + 3))
        def _():
            pltpu.make_async_copy(
                gbufs[b], shared_agg.at[ibd[p2].at[0]], semss[b]).wait()

        @pl.when(c + 2 < NCHUNK)
        def _():
            fetch_idx(c + 2, p2)

        @pl.when(c < NCHUNK)
        def _():
            wait_idx(c, p)
            pltpu.async_copy(xs_hbm.at[ibs[p].at[0]], gbufs[b], semgs[b])

        @pl.when(jnp.logical_and(c >= 2, c < NCHUNK + 2))
        def _():
            pltpu.make_async_copy(
                xs_hbm.at[ibs[p1].at[0]], gbufs[b2], semgs[b2]).wait()
            pltpu.async_copy(
                gbufs[b2], shared_agg.at[ibd[p1].at[0]], semss[b2], add=True)

    def body(jj, carry):
        for u in range(15):
            c = 15 * jj + u
            slot(c, u % 3, u % 5)
        return carry

    lax.fori_loop(0, (NCHUNK + 10) // 15, body, 0)
    plsc.subcore_barrier()

    @pl.when(sid % 2 == 0)
    def _():
        pltpu.sync_copy(shared_agg.at[span], out_hbm.at[cid, span])


def _prep_body(deg_ref, x_ref, xs_ref, dinv_ref, dnorm_ref):
    deg = (deg_ref[0] + deg_ref[1])[:NPAD]
    dinv = lax.rsqrt(deg + 1.0)
    dinv_ref[...] = dinv
    maxdeg = jnp.max(deg[:N])
    dnorm_ref[...] = deg / jnp.maximum(maxdeg, 1.0)
    xs_ref[:N] = x_ref[...] * dinv[:N]
    xs_ref[N:] = jnp.zeros((NPAD - N, DF), jnp.float32)


def _prep_call(deg3, x):
    return pl.pallas_call(
        _prep_body,
        out_shape=[
            jax.ShapeDtypeStruct((NPAD, DF), jnp.float32),
            jax.ShapeDtypeStruct((NPAD, 1), jnp.float32),
            jax.ShapeDtypeStruct((NPAD, 1), jnp.float32),
        ],
    )(deg3, x)


def _dense_body(agg_ref, xs_ref, dinv_ref, dnorm_ref, w1_ref, b1_ref, w2_ref,
                b2_ref, tf_ref, tc_ref, alpha_ref, gx_ref, gy_ref, bx_ref,
                by_ref, lwx_ref, lwy_ref, lb_ref, out_ref):
    f32 = jnp.float32
    agg = agg_ref[0] + agg_ref[1] + xs_ref[...]
    conv = dinv_ref[...] * agg
    h1 = jnp.maximum(
        jnp.dot(conv, w1_ref[...], preferred_element_type=f32) + b1_ref[...], 0.0)
    x2 = jnp.maximum(
        jnp.dot(conv, w2_ref[...], preferred_element_type=f32) + b2_ref[...], 0.0)

    tf = tf_ref[...]
    qf = jnp.mean(tf, axis=1)
    qf2 = jnp.mean(jnp.sum(tf * tf, axis=2), axis=1)
    sk = jnp.mean(tc_ref[...], axis=(1, 2))

    xx = jnp.sum(h1 * h1, axis=1, keepdims=True)
    cross = lax.dot_general(h1, qf, (((1,), (1,)), ((), ())),
                            preferred_element_type=f32)
    feat = xx + qf2[None, :] - 2.0 * cross
    struct = (dnorm_ref[...] - sk[None, :]) ** 2
    alpha = jax.nn.sigmoid(alpha_ref[0, 0])
    y = alpha * feat + (1.0 - alpha) * struct

    n = jnp.float32(N)
    m64 = lax.broadcasted_iota(jnp.int32, (NPAD, H), 0) < N
    m16 = lax.broadcasted_iota(jnp.int32, (NPAD, K), 0) < N
    mean_x = jnp.sum(jnp.where(m64, x2, 0.0), axis=0, keepdims=True) / n
    dx = jnp.where(m64, x2 - mean_x, 0.0)
    var_x = jnp.sum(dx * dx, axis=0, keepdims=True) / n
    mean_y = jnp.sum(jnp.where(m16, y, 0.0), axis=0, keepdims=True) / n
    dy = jnp.where(m16, y - mean_y, 0.0)
    var_y = jnp.sum(dy * dy, axis=0, keepdims=True) / n

    zx = (x2 - mean_x) * lax.rsqrt(var_x + 1e-5) * gx_ref[...] + bx_ref[...]
    zy = (y - mean_y) * lax.rsqrt(var_y + 1e-5) * gy_ref[...] + by_ref[...]
    res = (jnp.dot(zx, lwx_ref[...], preferred_element_type=f32)
           + jnp.dot(zy, lwy_ref[...], preferred_element_type=f32)
           + lb_ref[...])
    out_ref[...] = res[:N]


def _dense_call(*args):
    return pl.pallas_call(
        _dense_body,
        out_shape=jax.ShapeDtypeStruct((N, NC_OUT), jnp.float32),
    )(*args)


def kernel(x, edge_index, W1, b1, W2, b2, templates_F, templates_C,
           alpha_p, bn_gamma, bn_beta, lin_W, lin_b):
    f32 = jnp.float32
    pad = EPAD - E
    dummy = N + (jnp.arange(pad, dtype=jnp.int32) % 16)
    srcp = jnp.concatenate([edge_index[0], dummy]).reshape(NW, NCHUNK, CHUNK)
    dstp = jnp.concatenate([edge_index[1], dummy]).reshape(NW, NCHUNK, CHUNK)

    deg0, deg1 = _deg_kernel(dstp)
    deg3 = jnp.stack([deg0, deg1]).reshape(NCORES, NPAD_DEG, 1)
    xs, dinv, dnorm = _prep_call(deg3, x)
    zeros = jnp.zeros((NPAD, DF), f32)
    agg_parts = _agg_kernel(xs, srcp, dstp, zeros)

    out = _dense_call(
        agg_parts, xs, dinv, dnorm,
        W1, b1.reshape(1, H), W2, b2.reshape(1, H),
        templates_F, templates_C, alpha_p.reshape(1, 1),
        bn_gamma[:H].reshape(1, H), bn_gamma[H:].reshape(1, K),
        bn_beta[:H].reshape(1, H), bn_beta[H:].reshape(1, K),
        lin_W[:H], lin_W[H:], lin_b.reshape(1, NC_OUT),
    )
    return out


# speedup vs baseline: 2.0481x; 1.0315x over previous
import functools

import jax
import jax.numpy as jnp
import numpy as np
from jax import lax
from jax.experimental import pallas as pl
from jax.experimental.pallas import tpu as pltpu
from jax.experimental.pallas import tpu_sc as plsc

N = 10000
E = 320000
DF = 128
H = 64
K = 16
NC_OUT = 8

NCORES = 2
NSUB = 16
NW = NCORES * NSUB
CHUNK = 128
NCHUNK = 80
EPAD = NW * NCHUNK * CHUNK
NPAD_DEG = 10240
ROWS_PER_TILE = NPAD_DEG // NSUB
NPAD = 10048
ROWS_AGG = NPAD // NSUB

_mesh = plsc.VectorSubcoreMesh(core_axis_name="c", subcore_axis_name="s")


@functools.partial(
    pl.kernel,
    out_type=[jax.ShapeDtypeStruct((NPAD_DEG,), jnp.float32),
              jax.ShapeDtypeStruct((NPAD_DEG,), jnp.float32)],
    mesh=_mesh,
    scratch_types=[
        pltpu.VMEM((NCHUNK, CHUNK), jnp.int32),
        pltpu.VMEM((CHUNK,), jnp.float32),
        pltpu.VMEM((ROWS_PER_TILE,), jnp.float32),
        pltpu.VMEM_SHARED((NPAD_DEG,), jnp.float32),
        pltpu.SemaphoreType.DMA,
    ],
)
def _deg_kernel(dst_hbm, out0_hbm, out1_hbm, idx_v, ones_v, zrow_v, shared_deg, sem):
    cid = lax.axis_index("c")
    sid = lax.axis_index("s")
    wid = cid * NSUB + sid
    for i in range(CHUNK // 16):
        ones_v[pl.ds(16 * i, 16)] = jnp.full((16,), 1.0, jnp.float32)
    for i in range(ROWS_PER_TILE // 16):
        zrow_v[pl.ds(16 * i, 16)] = jnp.zeros((16,), jnp.float32)
    pltpu.sync_copy(zrow_v,
                    shared_deg.at[pl.ds(sid * ROWS_PER_TILE, ROWS_PER_TILE)])
    pltpu.sync_copy(dst_hbm.at[wid], idx_v)
    plsc.subcore_barrier()

    def body(j, carry):
        pltpu.async_copy(ones_v, shared_deg.at[idx_v.at[j]], sem, add=True)
        return carry

    lax.fori_loop(0, NCHUNK, body, 0)

    def drain(j, carry):
        pltpu.make_async_copy(ones_v, shared_deg.at[idx_v.at[j]], sem).wait()
        return carry

    lax.fori_loop(0, NCHUNK, drain, 0)
    plsc.subcore_barrier()
    tile_rows = pl.ds(sid * ROWS_PER_TILE, ROWS_PER_TILE)

    @pl.when(cid == 0)
    def _():
        pltpu.sync_copy(shared_deg.at[tile_rows], out0_hbm.at[tile_rows])

    @pl.when(cid == 1)
    def _():
        pltpu.sync_copy(shared_deg.at[tile_rows], out1_hbm.at[tile_rows])


@functools.partial(
    pl.kernel,
    out_type=jax.ShapeDtypeStruct((NCORES, NPAD, DF), jnp.float32),
    mesh=_mesh,
    scratch_types=(
        [pltpu.VMEM((1, CHUNK), jnp.int32)] * 10
        + [pltpu.VMEM((CHUNK, DF), jnp.float32)] * 3
        + [pltpu.SemaphoreType.DMA] * 11
        + [pltpu.VMEM_SHARED((NPAD, DF), jnp.float32)]
    ),
)
def _agg_kernel(xs_hbm, src_hbm, dst_hbm, out_hbm,
                ibs0, ibd0, ibs1, ibd1, ibs2, ibd2, ibs3, ibd3, ibs4, ibd4,
                gbuf0, gbuf1, gbuf2,
                semg0, semg1, semg2, sems0, sems1, sems2,
                semi0, semi1, semi2, semi3, semi4,
                shared_agg):
    cid = lax.axis_index("c")
    sid = lax.axis_index("s")
    wid = cid * NSUB + sid
    ibs = (ibs0, ibs1, ibs2, ibs3, ibs4)
    ibd = (ibd0, ibd1, ibd2, ibd3, ibd4)
    semi = (semi0, semi1, semi2, semi3, semi4)
    gbufs = (gbuf0, gbuf1, gbuf2)
    semgs = (semg0, semg1, semg2)
    semss = (sems0, sems1, sems2)
    span = pl.ds(pl.multiple_of((sid // 2) * (2 * ROWS_AGG), 8), 2 * ROWS_AGG)

    def zrow(r, carry):
        for gi in range(DF // 16):
            gbuf0[r, pl.ds(16 * gi, 16)] = jnp.zeros((16,), jnp.float32)
        return carry

    lax.fori_loop(0, CHUNK, zrow, 0)

    @pl.when(sid % 2 == 0)
    def _():
        base = pl.multiple_of((sid // 2) * (2 * ROWS_AGG), 8)
        for k in range(9):
            blk = CHUNK if k < 9 else 0
            pltpu.sync_copy(
                gbuf0.at[pl.ds(0, CHUNK if k < 9 else CHUNK)],
                shared_agg.at[pl.ds(base + k * CHUNK, CHUNK)])
        pltpu.sync_copy(gbuf0.at[pl.ds(0, 2 * ROWS_AGG - 9 * CHUNK)],
                        shared_agg.at[pl.ds(base + 9 * CHUNK,
                                            2 * ROWS_AGG - 9 * CHUNK)])

    plsc.subcore_barrier()

    def fetch_idx(c, p):
        pltpu.async_copy(src_hbm.at[wid, c], ibs[p].at[0], semi[p])
        pltpu.async_copy(dst_hbm.at[wid, c], ibd[p].at[0], semi[p])

    def wait_idx(c, p):
        pltpu.make_async_copy(src_hbm.at[wid, c], ibs[p].at[0], semi[p]).wait()
        pltpu.make_async_copy(dst_hbm.at[wid, c], ibd[p].at[0], semi[p]).wait()

    fetch_idx(0, 0)
    fetch_idx(1, 1)

    def slot(c, b, p):
        b2 = (b + 1) % 3
        p2 = (p + 2) % 5
        p1 = (p + 3) % 5

        @pl.when(jnp.logical_and(c >= 3, c < NCHUNK + 3))
        def _():
            pltpu.make_async_copy(
                gbufs[b], shared_agg.at[ibd[p2].at[0]], semss[b]).wait()

        @pl.when(c + 2 < NCHUNK)
        def _():
            fetch_idx(c + 2, p2)

        @pl.when(c < NCHUNK)
        def _():
            wait_idx(c, p)
            pltpu.async_copy(xs_hbm.at[ibs[p].at[0]], gbufs[b], semgs[b])

        @pl.when(jnp.logical_and(c >= 2, c < NCHUNK + 2))
        def _():
            pltpu.make_async_copy(
                xs_hbm.at[ibs[p1].at[0]], gbufs[b2], semgs[b2]).wait()
            pltpu.async_copy(
                gbufs[b2], shared_agg.at[ibd[p1].at[0]], semss[b2], add=True)

    def body(jj, carry):
        for u in range(15):
            c = 15 * jj + u
            slot(c, u % 3, u % 5)
        return carry

    lax.fori_loop(0, (NCHUNK + 10) // 15, body, 0)
    plsc.subcore_barrier()

    @pl.when(sid % 2 == 0)
    def _():
        pltpu.sync_copy(shared_agg.at[span], out_hbm.at[cid, span])


def _prep_body(deg_ref, x_ref, xs_ref, dinv_ref, dnorm_ref):
    deg = (deg_ref[0] + deg_ref[1])[:NPAD]
    dinv = lax.rsqrt(deg + 1.0)
    dinv_ref[...] = dinv
    maxdeg = jnp.max(deg[:N])
    dnorm_ref[...] = deg / jnp.maximum(maxdeg, 1.0)
    xs_ref[:N] = x_ref[...] * dinv[:N]
    xs_ref[N:] = jnp.zeros((NPAD - N, DF), jnp.float32)


def _prep_call(deg3, x):
    return pl.pallas_call(
        _prep_body,
        out_shape=[
            jax.ShapeDtypeStruct((NPAD, DF), jnp.float32),
            jax.ShapeDtypeStruct((NPAD, 1), jnp.float32),
            jax.ShapeDtypeStruct((NPAD, 1), jnp.float32),
        ],
    )(deg3, x)


def _dense_body(agg_ref, xs_ref, dinv_ref, dnorm_ref, w1_ref, b1_ref, w2_ref,
                b2_ref, tf_ref, tc_ref, alpha_ref, gx_ref, gy_ref, bx_ref,
                by_ref, lwx_ref, lwy_ref, lb_ref, out_ref):
    f32 = jnp.float32
    agg = agg_ref[0] + agg_ref[1] + xs_ref[...]
    conv = dinv_ref[...] * agg
    h1 = jnp.maximum(
        jnp.dot(conv, w1_ref[...], preferred_element_type=f32) + b1_ref[...], 0.0)
    x2 = jnp.maximum(
        jnp.dot(conv, w2_ref[...], preferred_element_type=f32) + b2_ref[...], 0.0)

    tf = tf_ref[...]
    qf = jnp.mean(tf, axis=1)
    qf2 = jnp.mean(jnp.sum(tf * tf, axis=2), axis=1)
    sk = jnp.mean(tc_ref[...], axis=(1, 2))

    xx = jnp.sum(h1 * h1, axis=1, keepdims=True)
    cross = lax.dot_general(h1, qf, (((1,), (1,)), ((), ())),
                            preferred_element_type=f32)
    feat = xx + qf2[None, :] - 2.0 * cross
    struct = (dnorm_ref[...] - sk[None, :]) ** 2
    alpha = jax.nn.sigmoid(alpha_ref[0, 0])
    y = alpha * feat + (1.0 - alpha) * struct

    n = jnp.float32(N)
    m64 = lax.broadcasted_iota(jnp.int32, (NPAD, H), 0) < N
    m16 = lax.broadcasted_iota(jnp.int32, (NPAD, K), 0) < N
    mean_x = jnp.sum(jnp.where(m64, x2, 0.0), axis=0, keepdims=True) / n
    dx = jnp.where(m64, x2 - mean_x, 0.0)
    var_x = jnp.sum(dx * dx, axis=0, keepdims=True) / n
    mean_y = jnp.sum(jnp.where(m16, y, 0.0), axis=0, keepdims=True) / n
    dy = jnp.where(m16, y - mean_y, 0.0)
    var_y = jnp.sum(dy * dy, axis=0, keepdims=True) / n

    zx = (x2 - mean_x) * lax.rsqrt(var_x + 1e-5) * gx_ref[...] + bx_ref[...]
    zy = (y - mean_y) * lax.rsqrt(var_y + 1e-5) * gy_ref[...] + by_ref[...]
    res = (jnp.dot(zx, lwx_ref[...], preferred_element_type=f32)
           + jnp.dot(zy, lwy_ref[...], preferred_element_type=f32)
           + lb_ref[...])
    out_ref[...] = res[:N]


def _dense_call(*args):
    return pl.pallas_call(
        _dense_body,
        out_shape=jax.ShapeDtypeStruct((N, NC_OUT), jnp.float32),
    )(*args)


def kernel(x, edge_index, W1, b1, W2, b2, templates_F, templates_C,
           alpha_p, bn_gamma, bn_beta, lin_W, lin_b):
    f32 = jnp.float32
    pad = EPAD - E
    dummy = N + (jnp.arange(pad, dtype=jnp.int32) % 16)
    srcp = jnp.concatenate([edge_index[0], dummy]).reshape(NW, NCHUNK, CHUNK)
    dstp = jnp.concatenate([edge_index[1], dummy]).reshape(NW, NCHUNK, CHUNK)

    deg0, deg1 = _deg_kernel(dstp)
    deg3 = jnp.stack([deg0, deg1]).reshape(NCORES, NPAD_DEG, 1)
    xs, dinv, dnorm = _prep_call(deg3, x)
    agg_parts = _agg_kernel(xs, srcp, dstp)

    out = _dense_call(
        agg_parts, xs, dinv, dnorm,
        W1, b1.reshape(1, H), W2, b2.reshape(1, H),
        templates_F, templates_C, alpha_p.reshape(1, 1),
        bn_gamma[:H].reshape(1, H), bn_gamma[H:].reshape(1, K),
        bn_beta[:H].reshape(1, H), bn_beta[H:].reshape(1, K),
        lin_W[:H], lin_W[H:], lin_b.reshape(1, NC_OUT),
    )
    return out
